# baseline (speedup 1.0000x reference)
import jax
import jax.numpy as jnp
from jax.experimental import pallas as pl

_S = 4
_K = 8192
_D = 32
_CHUNK = 2048
_NCHUNK = _K // _CHUNK


def _rq_body(z_ref, cb_ref, cbt_ref, zq_ref, loss_ref):
    z = z_ref[...]
    n_tokens = z.shape[0]
    residual = z
    recon = jnp.zeros_like(z)
    loss = jnp.zeros((1, 1), jnp.float32)
    inv_count = 1.0 / (n_tokens * _D)
    for s in range(_S):
        r2 = jnp.sum(residual * residual, axis=1, keepdims=True)
        best = jnp.full((n_tokens, 1), jnp.inf, jnp.float32)
        qbest = jnp.zeros((n_tokens, _D), jnp.float32)
        for c in range(_NCHUNK):
            cb = cb_ref[s, pl.ds(c * _CHUNK, _CHUNK), :]
            cbt = cbt_ref[s, :, pl.ds(c * _CHUNK, _CHUNK)]
            c2 = jnp.sum(cbt * cbt, axis=0, keepdims=True)
            dots = jax.lax.dot_general(
                residual.astype(jnp.bfloat16), cbt.astype(jnp.bfloat16),
                (((1,), (0,)), ((), ())),
                preferred_element_type=jnp.float32)
            dists = (r2 - 2.0 * dots) + c2
            m = jnp.min(dists, axis=1, keepdims=True)
            iota = jax.lax.broadcasted_iota(jnp.int32, dists.shape, 1)
            li = jnp.min(jnp.where(dists == m, iota, _CHUNK),
                         axis=1, keepdims=True)
            onehot = (iota == li).astype(jnp.float32)
            qc = jax.lax.dot_general(
                onehot, cb, (((1,), (0,)), ((), ())),
                preferred_element_type=jnp.float32,
                precision=jax.lax.Precision.HIGHEST)
            upd = m < best
            best = jnp.where(upd, m, best)
            qbest = jnp.where(upd, qc, qbest)
        err = qbest - residual
        sq = jnp.sum(err * err, axis=1, keepdims=True)
        loss = loss + jnp.sum(sq, axis=0, keepdims=True) * inv_count
        recon = recon + qbest
        residual = residual - qbest
    zq_ref[...] = z + (recon - z)
    loss_ref[...] = loss * 1.25


def kernel(z, codebooks):
    orig_shape = z.shape
    z_flat = z.reshape(-1, _D)
    n = z_flat.shape[0]
    cbt = codebooks.transpose(0, 2, 1)
    zq, loss = pl.pallas_call(
        _rq_body,
        out_shape=(
            jax.ShapeDtypeStruct((n, _D), jnp.float32),
            jax.ShapeDtypeStruct((1, 1), jnp.float32),
        ),
    )(z_flat, codebooks, cbt)
    return zq.reshape(orig_shape), loss[0, 0]


# speedup vs baseline: 1.6540x; 1.6540x over previous
import jax
import jax.numpy as jnp
from jax.experimental import pallas as pl
from jax.experimental.pallas import tpu as pltpu

_S = 4
_K = 8192
_D = 32
_CHUNK = 2048
_NCHUNK = _K // _CHUNK


def _rq_body(z_ref, cb_ref, cbt_ref, zq_ref, loss_ref,
             best_ref, qbest_ref, res_ref, recon_ref, lacc_ref):
    s = pl.program_id(0)
    c = pl.program_id(1)

    @pl.when(jnp.logical_and(s == 0, c == 0))
    def _init():
        res_ref[...] = z_ref[...]
        recon_ref[...] = jnp.zeros_like(recon_ref)
        lacc_ref[...] = jnp.zeros_like(lacc_ref)

    @pl.when(c == 0)
    def _stage_init():
        best_ref[...] = jnp.full_like(best_ref, jnp.inf)
        qbest_ref[...] = jnp.zeros_like(qbest_ref)

    residual = res_ref[...]
    r2 = jnp.sum(residual * residual, axis=1, keepdims=True)
    rb = residual.astype(jnp.bfloat16)
    cbt = cbt_ref[0]
    cb = cb_ref[0]

    c2 = jnp.sum(cbt * cbt, axis=0, keepdims=True)
    dots = jax.lax.dot_general(
        rb, cbt.astype(jnp.bfloat16), (((1,), (0,)), ((), ())),
        preferred_element_type=jnp.float32)
    dists = (r2 - 2.0 * dots) + c2
    m = jnp.min(dists, axis=1, keepdims=True)
    iota = jax.lax.broadcasted_iota(jnp.int32, dists.shape, 1)
    li = jnp.min(jnp.where(dists == m, iota, _CHUNK),
                 axis=1, keepdims=True)
    onehot = (iota == li).astype(jnp.bfloat16)

    hi = cb.astype(jnp.bfloat16)
    r1 = cb - hi.astype(jnp.float32)
    mid = r1.astype(jnp.bfloat16)
    lo = (r1 - mid.astype(jnp.float32)).astype(jnp.bfloat16)
    csplit = jnp.concatenate([hi, mid, lo], axis=1)
    qc3 = jax.lax.dot_general(
        onehot, csplit, (((1,), (0,)), ((), ())),
        preferred_element_type=jnp.float32)
    qc = (qc3[:, :_D] + qc3[:, _D:2 * _D]) + qc3[:, 2 * _D:]

    upd = m < best_ref[...]
    best_ref[...] = jnp.where(upd, m, best_ref[...])
    qbest_ref[...] = jnp.where(upd, qc, qbest_ref[...])

    @pl.when(c == _NCHUNK - 1)
    def _stage_fin():
        q = qbest_ref[...]
        err = q - res_ref[...]
        sq = jnp.sum(err * err, axis=1, keepdims=True)
        lacc_ref[...] += jnp.sum(sq, axis=0, keepdims=True) / (err.shape[0] * _D)
        recon_ref[...] += q
        res_ref[...] = res_ref[...] - q

    @pl.when(jnp.logical_and(s == _S - 1, c == _NCHUNK - 1))
    def _fin():
        z = z_ref[...]
        zq_ref[...] = z + (recon_ref[...] - z)
        loss_ref[...] = lacc_ref[...] * 1.25


def kernel(z, codebooks):
    orig_shape = z.shape
    z_flat = z.reshape(-1, _D)
    n = z_flat.shape[0]
    cbt = codebooks.transpose(0, 2, 1)
    zq, loss = pl.pallas_call(
        _rq_body,
        grid=(_S, _NCHUNK),
        in_specs=[
            pl.BlockSpec((n, _D), lambda s, c: (0, 0)),
            pl.BlockSpec((1, _CHUNK, _D), lambda s, c: (s, c, 0)),
            pl.BlockSpec((1, _D, _CHUNK), lambda s, c: (s, 0, c)),
        ],
        out_specs=(
            pl.BlockSpec((n, _D), lambda s, c: (0, 0)),
            pl.BlockSpec((1, 1), lambda s, c: (0, 0)),
        ),
        out_shape=(
            jax.ShapeDtypeStruct((n, _D), jnp.float32),
            jax.ShapeDtypeStruct((1, 1), jnp.float32),
        ),
        scratch_shapes=[
            pltpu.VMEM((n, 1), jnp.float32),
            pltpu.VMEM((n, _D), jnp.float32),
            pltpu.VMEM((n, _D), jnp.float32),
            pltpu.VMEM((n, _D), jnp.float32),
            pltpu.VMEM((1, 1), jnp.float32),
        ],
    )(z_flat, codebooks, cbt)
    return zq.reshape(orig_shape), loss[0, 0]


# speedup vs baseline: 2.0997x; 1.2694x over previous
import jax
import jax.numpy as jnp
from jax.experimental import pallas as pl
from jax.experimental.pallas import tpu as pltpu

_S = 4
_K = 8192
_D = 32
_CHUNK = 2048
_NCHUNK = _K // _CHUNK


def _rq_body(z_ref, cb_ref, cbt_ref, iota_ref, zq_ref, loss_ref,
             dbuf_ref, best_ref, qbest_ref, res_ref, recon_ref, lacc_ref):
    s = pl.program_id(0)
    c = pl.program_id(1)

    @pl.when(jnp.logical_and(s == 0, c == 0))
    def _init():
        res_ref[...] = z_ref[...]
        recon_ref[...] = jnp.zeros_like(recon_ref)
        lacc_ref[...] = jnp.zeros_like(lacc_ref)

    @pl.when(c == 0)
    def _stage_init():
        best_ref[...] = jnp.full_like(best_ref, jnp.inf)
        qbest_ref[...] = jnp.zeros_like(qbest_ref)

    @pl.when(c < _NCHUNK)
    def _produce():
        residual = res_ref[...]
        r2 = jnp.sum(residual * residual, axis=1, keepdims=True)
        rb2 = (residual + residual).astype(jnp.bfloat16)
        cbt = cbt_ref[0]
        c2 = jnp.sum(cbt * cbt, axis=0, keepdims=True)
        dots2 = jax.lax.dot_general(
            rb2, cbt.astype(jnp.bfloat16), (((1,), (0,)), ((), ())),
            preferred_element_type=jnp.float32)
        dbuf_ref[c % 2] = (r2 - dots2) + c2

    @pl.when(c > 0)
    def _consume():
        dists = dbuf_ref[(c - 1) % 2]
        m = jnp.min(dists, axis=1, keepdims=True)
        ib = iota_ref[...]
        li = jnp.min(jnp.where(dists == m, ib, float(_CHUNK)),
                     axis=1, keepdims=True)
        onehot = (ib == li).astype(jnp.bfloat16)

        cb = cb_ref[0]
        hi = cb.astype(jnp.bfloat16)
        r1 = cb - hi.astype(jnp.float32)
        mid = r1.astype(jnp.bfloat16)
        lo = (r1 - mid.astype(jnp.float32)).astype(jnp.bfloat16)
        csplit = jnp.concatenate([hi, mid, lo], axis=1)
        qc3 = jax.lax.dot_general(
            onehot, csplit, (((1,), (0,)), ((), ())),
            preferred_element_type=jnp.float32)
        qc = (qc3[:, :_D] + qc3[:, _D:2 * _D]) + qc3[:, 2 * _D:]

        upd = m < best_ref[...]
        best_ref[...] = jnp.where(upd, m, best_ref[...])
        qbest_ref[...] = jnp.where(upd, qc, qbest_ref[...])

    @pl.when(c == _NCHUNK)
    def _stage_fin():
        q = qbest_ref[...]
        err = q - res_ref[...]
        sq = jnp.sum(err * err, axis=1, keepdims=True)
        lacc_ref[...] += jnp.sum(sq, axis=0, keepdims=True) / (err.shape[0] * _D)
        recon_ref[...] += q
        res_ref[...] = res_ref[...] - q

    @pl.when(jnp.logical_and(s == _S - 1, c == _NCHUNK))
    def _fin():
        z = z_ref[...]
        zq_ref[...] = z + (recon_ref[...] - z)
        loss_ref[...] = lacc_ref[...] * 1.25


def kernel(z, codebooks):
    orig_shape = z.shape
    z_flat = z.reshape(-1, _D)
    n = z_flat.shape[0]
    cbt = codebooks.transpose(0, 2, 1)
    iota_row = jnp.arange(_CHUNK, dtype=jnp.float32).reshape(1, _CHUNK)
    zq, loss = pl.pallas_call(
        _rq_body,
        grid=(_S, _NCHUNK + 1),
        in_specs=[
            pl.BlockSpec((n, _D), lambda s, c: (0, 0)),
            pl.BlockSpec((1, _CHUNK, _D),
                         lambda s, c: (s, jnp.maximum(c - 1, 0), 0)),
            pl.BlockSpec((1, _D, _CHUNK),
                         lambda s, c: (s, 0, jnp.minimum(c, _NCHUNK - 1))),
            pl.BlockSpec((1, _CHUNK), lambda s, c: (0, 0)),
        ],
        out_specs=(
            pl.BlockSpec((n, _D), lambda s, c: (0, 0)),
            pl.BlockSpec((1, 1), lambda s, c: (0, 0)),
        ),
        out_shape=(
            jax.ShapeDtypeStruct((n, _D), jnp.float32),
            jax.ShapeDtypeStruct((1, 1), jnp.float32),
        ),
        scratch_shapes=[
            pltpu.VMEM((2, n, _CHUNK), jnp.float32),
            pltpu.VMEM((n, 1), jnp.float32),
            pltpu.VMEM((n, _D), jnp.float32),
            pltpu.VMEM((n, _D), jnp.float32),
            pltpu.VMEM((n, _D), jnp.float32),
            pltpu.VMEM((1, 1), jnp.float32),
        ],
    )(z_flat, codebooks, cbt, iota_row)
    return zq.reshape(orig_shape), loss[0, 0]
